# baseline (speedup 1.0000x reference)
import functools

import jax
import jax.numpy as jnp
from jax import lax
from jax.experimental import pallas as pl
from jax.experimental.pallas import tpu as pltpu
from jax.experimental.pallas import tpu_sc as plsc

N = 4194304
NUM_DATES = 500
NBINS = 512
SCALING = 12.0
ALPHA = 1.0

NC = 2
NS = 16
NW = NC * NS
PER_W = N // NW
BLK = 32768
NBLK = PER_W // BLK
LANES = 16
SUB = 512
NCHUNK = PER_W // SUB


_GRID = 8
_BELEMS = N // _GRID


def _vals_body(x_ref, t_ref, w_ref, o_ref):
    x = x_ref[...]
    sig = 1.0 / (1.0 + jnp.exp(x * (-SCALING)))
    o_ref[...] = w_ref[...] * t_ref[...] * sig


_vals_call = pl.pallas_call(
    _vals_body,
    grid=(_GRID,),
    in_specs=[pl.BlockSpec((_BELEMS,), lambda i: (i,))] * 3,
    out_specs=pl.BlockSpec((_BELEMS,), lambda i: (i,)),
    out_shape=jax.ShapeDtypeStruct((N,), jnp.float32),
)


def _make_sc_bincount(n, nw, blk, nbins):
    per_w = n // nw
    nblk = per_w // blk
    nchunk = per_w // SUB
    mesh = plsc.VectorSubcoreMesh(
        core_axis_name="c", subcore_axis_name="s", num_cores=NC,
        num_subcores=NS)

    @functools.partial(
        pl.kernel,
        out_type=jax.ShapeDtypeStruct((nw, nbins), jnp.float32),
        mesh=mesh,
        compiler_params=pltpu.CompilerParams(needs_layout_passes=False),
        scratch_types=[
            pltpu.VMEM((2, blk), jnp.float32),
            pltpu.VMEM((LANES, nbins + 1), jnp.float32),
            pltpu.VMEM((128,), jnp.int32),
            pltpu.VMEM((nchunk + LANES + 8,), jnp.int32),
            pltpu.VMEM((SUB,), jnp.int32),
            pltpu.SemaphoreType.DMA,
            pltpu.SemaphoreType.DMA,
            pltpu.SemaphoreType.DMA,
        ],
    )
    def sc_bincount(v_hbm, d_hbm, out_hbm, vb, bins, gidx, samp, dslow,
                    sem0, sem1, semg):
        wid = lax.axis_index("s") * NC + lax.axis_index("c")
        base = wid * per_w
        chunk0 = wid * nchunk
        sems = (sem0, sem1)

        def vcopy(g, slot):
            off = base + g * blk
            return pltpu.make_async_copy(
                v_hbm.at[pl.ds(off, blk)], vb.at[slot], sems[slot])

        iota = lax.iota(jnp.int32, LANES)
        nga = nchunk // 128

        def fill_idx(j0):
            for i in range(128 // LANES):
                gidx[pl.ds(i * LANES, LANES)] = (
                    (chunk0 + j0 + i * LANES) * SUB + iota * SUB)

        vcopy(0, 0).start()
        for a in range(nga):
            fill_idx(a * 128)
            pltpu.make_async_copy(d_hbm.at[gidx],
                                  samp.at[pl.ds(a * 128, 128)], semg).start()
            pltpu.make_async_copy(d_hbm.at[gidx],
                                  samp.at[pl.ds(a * 128, 128)], semg).wait()
        lastpos = jnp.minimum((chunk0 + nchunk) * SUB, n - 1)
        gidx[pl.ds(0, LANES)] = jnp.full((LANES,), lastpos, jnp.int32)
        pltpu.make_async_copy(d_hbm.at[gidx.at[pl.ds(0, 8)]],
                              samp.at[pl.ds(nchunk, 8)], semg).start()
        pltpu.make_async_copy(d_hbm.at[gidx.at[pl.ds(0, 8)]],
                              samp.at[pl.ds(nchunk, 8)], semg).wait()

        zeros = jnp.zeros((LANES,), jnp.float32)

        def zbody(i, _):
            o = i * LANES
            for r in range(LANES):
                bins[r, pl.ds(o, LANES)] = zeros
            return 0

        lax.fori_loop(0, nbins // LANES, zbody, 0)

        for g in range(nblk):
            slot = g % 2
            if g + 1 < nblk:
                vcopy(g + 1, 1 - slot).start()
            vcopy(g, slot).wait()

            def sub(c, _):
                o0 = c * SUB
                cg = g * (blk // SUB) + c
                sv = samp[pl.ds(cg, LANES)]
                s0 = sv[0]
                s1 = sv[1]

                @pl.when(s0 == s1)
                def _fast():
                    accs = [jnp.zeros((LANES,), jnp.float32)
                            for _ in range(4)]
                    for u in range(SUB // LANES):
                        o = o0 + u * LANES
                        vv = vb[slot, pl.ds(o, LANES)]
                        accs[u % 4] = accs[u % 4] + vv
                    acc = (accs[0] + accs[1]) + (accs[2] + accs[3])
                    dsplat = jnp.full((LANES,), s0, jnp.int32)
                    plsc.addupdate_scatter(bins, [iota, dsplat], acc)

                @pl.when(s0 != s1)
                def _slow():
                    pltpu.sync_copy(
                        d_hbm.at[pl.ds(base + g * blk + o0, SUB)], dslow)

                    @plsc.parallel_loop(0, SUB // LANES, unroll=4)
                    def sbody(u):
                        o = o0 + u * LANES
                        vv = vb[slot, pl.ds(o, LANES)]
                        dv = dslow[pl.ds(u * LANES, LANES)]
                        plsc.addupdate_scatter(bins, [iota, dv], vv)

                return 0

            lax.fori_loop(0, blk // SUB, sub, 0)

        def mbody(i, _):
            o = i * LANES
            acc = bins[0, pl.ds(o, LANES)]
            for r in range(1, LANES):
                acc = acc + bins[r, pl.ds(o, LANES)]
            bins[0, pl.ds(o, LANES)] = acc
            return 0

        lax.fori_loop(0, nbins // LANES, mbody, 0)
        pltpu.sync_copy(bins.at[0, pl.ds(0, nbins)], out_hbm.at[wid])

    return sc_bincount


_sc_bincount_full = _make_sc_bincount(N, NW, BLK, NBINS)


def _finalize_body(bins_ref, out_ref):
    pi = jnp.sum(bins_ref[...], axis=0, keepdims=True)
    total = jnp.sum(pi)
    ssq = jnp.sum(pi * pi)
    out_ref[0, 0] = -(ALPHA * total * total) / ssq


_finalize = pl.pallas_call(
    _finalize_body,
    out_shape=jax.ShapeDtypeStruct((1, 1), jnp.float32),
    in_specs=[pl.BlockSpec(memory_space=pltpu.VMEM)],
    out_specs=pl.BlockSpec(memory_space=pltpu.SMEM),
)


def kernel(inputs, targets, weight, date):
    date_i = date.astype(jnp.int32)
    vals = _vals_call(inputs, targets, weight)
    part = _sc_bincount_full(vals, date_i)
    return _finalize(part)[0, 0]


# speedup vs baseline: 1.0830x; 1.0830x over previous
import functools

import jax
import jax.numpy as jnp
from jax import lax
from jax.experimental import pallas as pl
from jax.experimental.pallas import tpu as pltpu
from jax.experimental.pallas import tpu_sc as plsc

N = 4194304
NUM_DATES = 500
NBINS = 512
SCALING = 12.0
ALPHA = 1.0

NC = 2
NS = 16
NW = NC * NS
PER_W = N // NW
BLK = 16384
LANES = 16
SUB = 512


_GRID = 8
_BELEMS = N // _GRID
_HALF = N // 2
_HGRID = _GRID // 2


def _vals_body(x_ref, t_ref, w_ref, o_ref):
    x = x_ref[...]
    sig = 1.0 / (1.0 + jnp.exp(x * (-SCALING)))
    o_ref[...] = w_ref[...] * t_ref[...] * sig


def _make_vals(half):
    base = half * _HGRID
    return pl.pallas_call(
        _vals_body,
        grid=(_HGRID,),
        in_specs=[pl.BlockSpec((_BELEMS,), lambda i: (i + base,))] * 3,
        out_specs=pl.BlockSpec((_BELEMS,), lambda i: (i,)),
        out_shape=jax.ShapeDtypeStruct((_HALF,), jnp.float32),
    )


_vals_calls = (_make_vals(0), _make_vals(1))


def _make_sc_bincount(n, nw, blk, nbins, date_base):
    per_w = n // nw
    nblk = per_w // blk
    mesh = plsc.VectorSubcoreMesh(
        core_axis_name="c", subcore_axis_name="s", num_cores=NC,
        num_subcores=NS)

    @functools.partial(
        pl.kernel,
        out_type=jax.ShapeDtypeStruct((nw, nbins), jnp.float32),
        mesh=mesh,
        compiler_params=pltpu.CompilerParams(needs_layout_passes=False),
        scratch_types=[
            pltpu.VMEM((2, blk), jnp.float32),
            pltpu.VMEM((2, blk), jnp.int32),
            pltpu.VMEM((LANES, nbins + 1), jnp.float32),
            pltpu.SemaphoreType.DMA,
            pltpu.SemaphoreType.DMA,
        ],
    )
    def sc_bincount(v_hbm, d_hbm, out_hbm, vb, db, bins, sem0, sem1):
        wid = lax.axis_index("s") * NC + lax.axis_index("c")
        base = wid * per_w
        sems = (sem0, sem1)

        def copies(g, slot):
            off = base + g * blk
            sem = sems[slot]
            return (
                pltpu.make_async_copy(v_hbm.at[pl.ds(off, blk)], vb.at[slot], sem),
                pltpu.make_async_copy(d_hbm.at[pl.ds(date_base + off, blk)],
                                      db.at[slot], sem),
            )

        def start(g, slot):
            for c in copies(g, slot):
                c.start()

        def wait(g, slot):
            for c in copies(g, slot):
                c.wait()

        zeros = jnp.zeros((LANES,), jnp.float32)

        def zbody(i, _):
            o = i * LANES
            for r in range(LANES):
                bins[r, pl.ds(o, LANES)] = zeros
            return 0

        lax.fori_loop(0, nbins // LANES, zbody, 0)
        lane = lax.iota(jnp.int32, LANES)

        start(0, 0)
        for g in range(nblk):
            slot = g % 2
            if g + 1 < nblk:
                start(g + 1, 1 - slot)
            wait(g, slot)

            def sub(c, _):
                o0 = c * SUB
                d0v = db[slot, pl.ds(o0, LANES)]
                d1v = db[slot, pl.ds(o0 + SUB - LANES, LANES)]
                d0 = d0v[0]
                d1 = d1v[LANES - 1]

                @pl.when(d0 == d1)
                def _fast():
                    accs = [jnp.zeros((LANES,), jnp.float32)
                            for _ in range(4)]
                    for u in range(SUB // LANES):
                        o = o0 + u * LANES
                        vv = vb[slot, pl.ds(o, LANES)]
                        accs[u % 4] = accs[u % 4] + vv
                    acc = (accs[0] + accs[1]) + (accs[2] + accs[3])
                    dsplat = jnp.full((LANES,), d0, jnp.int32)
                    plsc.addupdate_scatter(bins, [lane, dsplat], acc)

                @pl.when(d0 != d1)
                def _slow():
                    @plsc.parallel_loop(0, SUB // LANES, unroll=4)
                    def sbody(u):
                        o = o0 + u * LANES
                        vv = vb[slot, pl.ds(o, LANES)]
                        dv = db[slot, pl.ds(o, LANES)]
                        plsc.addupdate_scatter(bins, [lane, dv], vv)

                return 0

            lax.fori_loop(0, blk // SUB, sub, 0)

        def mbody(i, _):
            o = i * LANES
            acc = bins[0, pl.ds(o, LANES)]
            for r in range(1, LANES):
                acc = acc + bins[r, pl.ds(o, LANES)]
            bins[0, pl.ds(o, LANES)] = acc
            return 0

        lax.fori_loop(0, nbins // LANES, mbody, 0)
        pltpu.sync_copy(bins.at[0, pl.ds(0, nbins)], out_hbm.at[wid])

    return sc_bincount


_sc_bincounts = (_make_sc_bincount(_HALF, NW, BLK, NBINS, 0),
                 _make_sc_bincount(_HALF, NW, BLK, NBINS, _HALF))


def _finalize_body(b0_ref, b1_ref, out_ref):
    pi = (jnp.sum(b0_ref[...], axis=0, keepdims=True)
          + jnp.sum(b1_ref[...], axis=0, keepdims=True))
    total = jnp.sum(pi)
    ssq = jnp.sum(pi * pi)
    out_ref[0, 0] = -(ALPHA * total * total) / ssq


_finalize = pl.pallas_call(
    _finalize_body,
    out_shape=jax.ShapeDtypeStruct((1, 1), jnp.float32),
    in_specs=[pl.BlockSpec(memory_space=pltpu.VMEM)] * 2,
    out_specs=pl.BlockSpec(memory_space=pltpu.SMEM),
)


def kernel(inputs, targets, weight, date):
    date_i = date.astype(jnp.int32)
    v0 = _vals_calls[0](inputs, targets, weight)
    p0 = _sc_bincounts[0](v0, date_i)
    v1 = _vals_calls[1](inputs, targets, weight)
    p1 = _sc_bincounts[1](v1, date_i)
    return _finalize(p0, p1)[0, 0]
